# baseline (speedup 1.0000x reference)
import functools

import jax
import jax.numpy as jnp
from jax import lax
from jax.experimental import pallas as pl
from jax.experimental.pallas import tpu as pltpu
from jax.experimental.pallas import tpu_sc as plsc

_SEQ = 8192
_D = 768
_B = 4
_T = _SEQ * _D
_N = _B * _T

_NC = 2
_NS = 16
_NW = _NC * _NS
_TW = _T // _NW
_C = 16384
_NJ = _TW // _C
_LANES = 16
_VPC = _C // _LANES


def _sc_body(in_hbm, tab_hbm, out_hbm, tab_v, in_v, acc_v):
    wid = lax.axis_index("s") * _NC + lax.axis_index("c")
    tbase = wid * _TW

    def chunk(j, _):
        toff = tbase + j * _C
        pltpu.sync_copy(tab_hbm.at[pl.ds(toff, _C)], tab_v)
        for b in range(_B):
            ioff = b * _T + toff
            pltpu.sync_copy(in_hbm.at[pl.ds(ioff, _C)], in_v)

            def add16(i, _):
                sl = pl.ds(i * _LANES, _LANES)
                acc_v[sl] = in_v[sl] + tab_v[sl]
                return 0

            lax.fori_loop(0, _VPC, add16, 0, unroll=8)
            pltpu.sync_copy(acc_v, out_hbm.at[pl.ds(ioff, _C)])
        return 0

    lax.fori_loop(0, _NJ, chunk, 0)


@jax.jit
def kernel(inputs, pos_table):
    mesh = plsc.VectorSubcoreMesh(core_axis_name="c", subcore_axis_name="s")
    k = pl.kernel(
        _sc_body,
        out_type=jax.ShapeDtypeStruct((_N,), jnp.float32),
        mesh=mesh,
        scratch_types=[
            pltpu.VMEM((_C,), jnp.float32),
            pltpu.VMEM((_C,), jnp.float32),
            pltpu.VMEM((_C,), jnp.float32),
        ],
    )
    out = k(jnp.reshape(inputs, (_N,)), jnp.reshape(pos_table, (_T,)))
    return jnp.reshape(out, (_B, _SEQ, _D))


# speedup vs baseline: 1.0904x; 1.0904x over previous
import functools

import jax
import jax.numpy as jnp
from jax import lax
from jax.experimental import pallas as pl
from jax.experimental.pallas import tpu as pltpu
from jax.experimental.pallas import tpu_sc as plsc

_SEQ = 8192
_D = 768
_B = 4
_T = _SEQ * _D
_N = _B * _T

_NC = 2
_NS = 16
_NW = _NC * _NS
_TW = _T // _NW
_C = 16384
_NJ = _TW // _C
_NITEMS = _NJ * _B
_LANES = 16
_VPC = _C // _LANES


def _sc_body(in_hbm, tab_hbm, out_hbm, tab_v, in_v, out_v,
             tab_sem, in_sem, out_sem):
    wid = lax.axis_index("s") * _NC + lax.axis_index("c")
    tbase = wid * _TW

    def tab_copy(j, jp):
        return pltpu.make_async_copy(
            tab_hbm.at[pl.ds(tbase + j * _C, _C)], tab_v.at[jp], tab_sem.at[jp])

    def in_copy(t, p):
        ioff = (t % _B) * _T + tbase + (t // _B) * _C
        return pltpu.make_async_copy(
            in_hbm.at[pl.ds(ioff, _C)], in_v.at[p], in_sem.at[p])

    def out_copy(t, p):
        ioff = (t % _B) * _T + tbase + (t // _B) * _C
        return pltpu.make_async_copy(
            out_v.at[p], out_hbm.at[pl.ds(ioff, _C)], out_sem.at[p])

    tab_copy(0, 0).start()
    in_copy(0, 0).start()

    def item(t, _):
        j = t // _B
        b = t % _B
        p = t % 2
        jp = j % 2

        @pl.when(t + 1 < _NITEMS)
        def _():
            in_copy(t + 1, (t + 1) % 2).start()

        @pl.when((b == 0) & (j + 1 < _NJ))
        def _():
            tab_copy(j + 1, (j + 1) % 2).start()

        in_copy(t, p).wait()

        @pl.when(b == 0)
        def _():
            tab_copy(j, jp).wait()

        @pl.when(t >= 2)
        def _():
            out_copy(t - 2, p).wait()

        def add16(i, _):
            sl = pl.ds(i * _LANES, _LANES)
            out_v[p, sl] = in_v[p, sl] + tab_v[jp, sl]
            return 0

        lax.fori_loop(0, _VPC, add16, 0, unroll=8)
        out_copy(t, p).start()
        return 0

    lax.fori_loop(0, _NITEMS, item, 0)

    out_copy(_NITEMS - 2, 0).wait()
    out_copy(_NITEMS - 1, 1).wait()


@jax.jit
def kernel(inputs, pos_table):
    mesh = plsc.VectorSubcoreMesh(core_axis_name="c", subcore_axis_name="s")
    k = pl.kernel(
        _sc_body,
        out_type=jax.ShapeDtypeStruct((_N,), jnp.float32),
        mesh=mesh,
        scratch_types=[
            pltpu.VMEM((2, _C), jnp.float32),
            pltpu.VMEM((2, _C), jnp.float32),
            pltpu.VMEM((2, _C), jnp.float32),
            pltpu.SemaphoreType.DMA((2,)),
            pltpu.SemaphoreType.DMA((2,)),
            pltpu.SemaphoreType.DMA((2,)),
        ],
    )
    out = k(jnp.reshape(inputs, (_N,)), jnp.reshape(pos_table, (_T,)))
    return jnp.reshape(out, (_B, _SEQ, _D))


# speedup vs baseline: 1.6914x; 1.5512x over previous
import functools

import jax
import jax.numpy as jnp
from jax import lax
from jax.experimental import pallas as pl
from jax.experimental.pallas import tpu as pltpu
from jax.experimental.pallas import tpu_sc as plsc

_SEQ = 8192
_D = 768
_B = 4
_T = _SEQ * _D
_N = _B * _T

_NC = 2
_NS = 16
_NW = _NC * _NS
_TW = _T // _NW
_C = 16384
_NJ = _TW // _C
_NITEMS = _NJ * _B
_LANES = 16
_VPC = _C // _LANES


def _sc_body(in_hbm, tab_hbm, out_hbm, tab_v, in_v, out_v,
             tab_sem, in_sem, out_sem):
    wid = lax.axis_index("s") * _NC + lax.axis_index("c")
    tbase = wid * _TW

    def tab_copy(j, jp):
        return pltpu.make_async_copy(
            tab_hbm.at[pl.ds(tbase + j * _C, _C)], tab_v.at[jp], tab_sem.at[jp])

    def in_copy(t, p):
        ioff = (t % _B) * _T + tbase + (t // _B) * _C
        return pltpu.make_async_copy(
            in_hbm.at[pl.ds(ioff, _C)], in_v.at[p], in_sem.at[p])

    def out_copy(t, p):
        ioff = (t % _B) * _T + tbase + (t // _B) * _C
        return pltpu.make_async_copy(
            out_v.at[p], out_hbm.at[pl.ds(ioff, _C)], out_sem.at[p])

    tab_copy(0, 0).start()
    in_copy(0, 0).start()

    def item(t, _):
        j = t // _B
        b = t % _B
        p = t % 2
        jp = j % 2

        @pl.when(t + 1 < _NITEMS)
        def _():
            in_copy(t + 1, (t + 1) % 2).start()

        @pl.when((b == 0) & (j + 1 < _NJ))
        def _():
            tab_copy(j + 1, (j + 1) % 2).start()

        in_copy(t, p).wait()

        @pl.when(b == 0)
        def _():
            tab_copy(j, jp).wait()

        @pl.when(t >= 2)
        def _():
            out_copy(t - 2, p).wait()

        @plsc.parallel_loop(0, _C, step=_LANES, unroll=8)
        def _(i):
            sl = pl.ds(i, _LANES)
            out_v[p, sl] = in_v[p, sl] + tab_v[jp, sl]
        out_copy(t, p).start()
        return 0

    lax.fori_loop(0, _NITEMS, item, 0)

    out_copy(_NITEMS - 2, 0).wait()
    out_copy(_NITEMS - 1, 1).wait()


@jax.jit
def kernel(inputs, pos_table):
    mesh = plsc.VectorSubcoreMesh(core_axis_name="c", subcore_axis_name="s")
    k = pl.kernel(
        _sc_body,
        out_type=jax.ShapeDtypeStruct((_N,), jnp.float32),
        mesh=mesh,
        scratch_types=[
            pltpu.VMEM((2, _C), jnp.float32),
            pltpu.VMEM((2, _C), jnp.float32),
            pltpu.VMEM((2, _C), jnp.float32),
            pltpu.SemaphoreType.DMA((2,)),
            pltpu.SemaphoreType.DMA((2,)),
            pltpu.SemaphoreType.DMA((2,)),
        ],
    )
    out = k(jnp.reshape(inputs, (_N,)), jnp.reshape(pos_table, (_T,)))
    return jnp.reshape(out, (_B, _SEQ, _D))


# speedup vs baseline: 5.5124x; 3.2590x over previous
import jax
import jax.numpy as jnp
from jax import lax
from jax.experimental import pallas as pl
from jax.experimental.pallas import tpu as pltpu
from jax.experimental.pallas import tpu_sc as plsc

_SEQ = 8192
_D = 768
_B = 4

_NC = 2
_NS = 16
_NW = _NC * _NS
_ROWS_W = _SEQ // _NW
_R = 16
_NJ = _ROWS_W // _R
_NITEMS = _NJ * _B
_LANES = 16


def _sc_body(in_hbm, tab_hbm, out_hbm, tab_v, in_v, out_v,
             tab_sem, in_sem, out_sem):
    wid = lax.axis_index("s") * _NC + lax.axis_index("c")
    rbase = wid * _ROWS_W

    def tab_copy(j, jp):
        return pltpu.make_async_copy(
            tab_hbm.at[pl.ds(rbase + j * _R, _R)], tab_v.at[jp], tab_sem.at[jp])

    def in_copy(t, p):
        r0 = rbase + (t // _B) * _R
        return pltpu.make_async_copy(
            in_hbm.at[t % _B, pl.ds(r0, _R)], in_v.at[p], in_sem.at[p])

    def out_copy(t, p):
        r0 = rbase + (t // _B) * _R
        return pltpu.make_async_copy(
            out_v.at[p], out_hbm.at[t % _B, pl.ds(r0, _R)], out_sem.at[p])

    tab_copy(0, 0).start()
    in_copy(0, 0).start()

    def item(t, _):
        j = t // _B
        b = t % _B
        p = t % 2
        jp = j % 2

        @pl.when(t + 1 < _NITEMS)
        def _():
            in_copy(t + 1, (t + 1) % 2).start()

        @pl.when((b == 0) & (j + 1 < _NJ))
        def _():
            tab_copy(j + 1, (j + 1) % 2).start()

        in_copy(t, p).wait()

        @pl.when(b == 0)
        def _():
            tab_copy(j, jp).wait()

        @pl.when(t >= 2)
        def _():
            out_copy(t - 2, p).wait()

        @plsc.parallel_loop(0, _R, step=1)
        def _(r):
            for c in range(0, _D, _LANES):
                sl = pl.ds(c, _LANES)
                out_v[p, r, sl] = in_v[p, r, sl] + tab_v[jp, r, sl]

        out_copy(t, p).start()
        return 0

    lax.fori_loop(0, _NITEMS, item, 0)

    out_copy(_NITEMS - 2, 0).wait()
    out_copy(_NITEMS - 1, 1).wait()


@jax.jit
def kernel(inputs, pos_table):
    mesh = plsc.VectorSubcoreMesh(core_axis_name="c", subcore_axis_name="s")
    k = pl.kernel(
        _sc_body,
        out_type=jax.ShapeDtypeStruct((_B, _SEQ, _D), jnp.float32),
        mesh=mesh,
        scratch_types=[
            pltpu.VMEM((2, _R, _D), jnp.float32),
            pltpu.VMEM((2, _R, _D), jnp.float32),
            pltpu.VMEM((2, _R, _D), jnp.float32),
            pltpu.SemaphoreType.DMA((2,)),
            pltpu.SemaphoreType.DMA((2,)),
            pltpu.SemaphoreType.DMA((2,)),
        ],
    )
    return k(inputs, pos_table)
